# baseline (speedup 1.0000x reference)
import jax
import jax.numpy as jnp
from jax.experimental import pallas as pl
from jax.experimental.pallas import tpu as pltpu

_B = 32
_CHUNK = 128
_CANVAS = 8192
_D = 256
_SPLIT = 2
_ROWS = _CANVAS // _SPLIT
_ALIGN = 8
_WIN = _CHUNK + _ALIGN


def _cache_update_kernel(index_ref, value_ref, cache_ref, out_ref,
                         win_ref, copy_sem, win_sem):
    idx = index_ref[0]
    base = pl.multiple_of((idx // _ALIGN) * _ALIGN, _ALIGN)
    off = idx - base

    win_in = pltpu.make_async_copy(
        cache_ref.at[:, pl.ds(base, _WIN), :], win_ref, win_sem)
    win_in.start()

    for b in range(_B):
        for s in range(_SPLIT):
            pltpu.make_async_copy(
                cache_ref.at[b, pl.ds(s * _ROWS, _ROWS), :],
                out_ref.at[b, pl.ds(s * _ROWS, _ROWS), :],
                copy_sem,
            ).start()

    win_in.wait()
    padded = jnp.concatenate(
        [value_ref[...], jnp.zeros((_B, _WIN - _CHUNK, _D), jnp.float32)],
        axis=1)
    rolled = pltpu.roll(padded, off, axis=1)
    r = jax.lax.broadcasted_iota(jnp.int32, (_B, _WIN, _D), 1)
    mask = (r >= off) & (r < off + _CHUNK)
    win_ref[...] = jnp.where(mask, rolled, win_ref[...])

    for b in range(_B):
        for s in range(_SPLIT):
            pltpu.make_async_copy(
                cache_ref.at[b, pl.ds(s * _ROWS, _ROWS), :],
                out_ref.at[b, pl.ds(s * _ROWS, _ROWS), :],
                copy_sem,
            ).wait()

    win_out = pltpu.make_async_copy(
        win_ref, out_ref.at[:, pl.ds(base, _WIN), :], win_sem)
    win_out.start()
    win_out.wait()


def kernel(value, index, cache):
    return pl.pallas_call(
        _cache_update_kernel,
        out_shape=jax.ShapeDtypeStruct((_B, _CANVAS, _D), cache.dtype),
        in_specs=[
            pl.BlockSpec(memory_space=pltpu.SMEM),
            pl.BlockSpec(memory_space=pltpu.VMEM),
            pl.BlockSpec(memory_space=pl.ANY),
        ],
        out_specs=pl.BlockSpec(memory_space=pl.ANY),
        scratch_shapes=[
            pltpu.VMEM((_B, _WIN, _D), jnp.float32),
            pltpu.SemaphoreType.DMA,
            pltpu.SemaphoreType.DMA,
        ],
    )(index, value, cache)


# speedup vs baseline: 43.9458x; 43.9458x over previous
import jax
import jax.numpy as jnp
from jax.experimental import pallas as pl
from jax.experimental.pallas import tpu as pltpu

_B = 32
_CHUNK = 128
_CANVAS = 8192
_D = 256
_RB = 2048
_ALIGN = 8
_WIN = _CHUNK + _ALIGN


def _cache_update_kernel(index_ref, value_ref, in_ref, out_ref):
    j = pl.program_id(1)
    start = j * _RB
    idx = index_ref[0]

    out_ref[...] = in_ref[...]

    overlap = jnp.logical_and(idx < start + _RB, idx + _CHUNK > start)

    @pl.when(overlap)
    def _merge():
        sh = idx - start
        p_raw = jnp.clip((sh // _ALIGN) * _ALIGN, 0, _RB - _WIN)
        p = pl.multiple_of(p_raw, _ALIGN)
        off = sh - p_raw
        padded = jnp.concatenate(
            [value_ref[...],
             jnp.zeros((1, _WIN - _CHUNK, _D), jnp.float32)], axis=1)
        rolled = pltpu.roll(padded, jnp.remainder(off, _WIN), axis=1)
        r = jax.lax.broadcasted_iota(jnp.int32, (1, _WIN, _D), 1)
        mask = jnp.logical_and(r >= off, r < off + _CHUNK)
        sub = in_ref[:, pl.ds(p, _WIN), :]
        out_ref[:, pl.ds(p, _WIN), :] = jnp.where(mask, rolled, sub)


def kernel(value, index, cache):
    grid_spec = pltpu.PrefetchScalarGridSpec(
        num_scalar_prefetch=1,
        grid=(_B, _CANVAS // _RB),
        in_specs=[
            pl.BlockSpec((1, _CHUNK, _D), lambda b, j, idx: (b, 0, 0)),
            pl.BlockSpec((1, _RB, _D), lambda b, j, idx: (b, j, 0)),
        ],
        out_specs=pl.BlockSpec((1, _RB, _D), lambda b, j, idx: (b, j, 0)),
    )
    return pl.pallas_call(
        _cache_update_kernel,
        grid_spec=grid_spec,
        out_shape=jax.ShapeDtypeStruct((_B, _CANVAS, _D), cache.dtype),
    )(index, value, cache)


# speedup vs baseline: 47.8847x; 1.0896x over previous
import jax
import jax.numpy as jnp
from jax.experimental import pallas as pl
from jax.experimental.pallas import tpu as pltpu

_B = 32
_CHUNK = 128
_CANVAS = 8192
_D = 256
_RB = 4096
_ALIGN = 8
_WIN = _CHUNK + _ALIGN


def _cache_update_kernel(index_ref, value_ref, in_ref, out_ref):
    j = pl.program_id(1)
    start = j * _RB
    idx = index_ref[0]

    out_ref[...] = in_ref[...]

    overlap = jnp.logical_and(idx < start + _RB, idx + _CHUNK > start)

    @pl.when(overlap)
    def _merge():
        sh = idx - start
        p_raw = jnp.clip((sh // _ALIGN) * _ALIGN, 0, _RB - _WIN)
        p = pl.multiple_of(p_raw, _ALIGN)
        off = sh - p_raw
        padded = jnp.concatenate(
            [value_ref[...],
             jnp.zeros((1, _WIN - _CHUNK, _D), jnp.float32)], axis=1)
        rolled = pltpu.roll(padded, jnp.remainder(off, _WIN), axis=1)
        r = jax.lax.broadcasted_iota(jnp.int32, (1, _WIN, _D), 1)
        mask = jnp.logical_and(r >= off, r < off + _CHUNK)
        sub = in_ref[:, pl.ds(p, _WIN), :]
        out_ref[:, pl.ds(p, _WIN), :] = jnp.where(mask, rolled, sub)


def kernel(value, index, cache):
    grid_spec = pltpu.PrefetchScalarGridSpec(
        num_scalar_prefetch=1,
        grid=(_B, _CANVAS // _RB),
        in_specs=[
            pl.BlockSpec((1, _CHUNK, _D), lambda b, j, idx: (b, 0, 0)),
            pl.BlockSpec((1, _RB, _D), lambda b, j, idx: (b, j, 0)),
        ],
        out_specs=pl.BlockSpec((1, _RB, _D), lambda b, j, idx: (b, j, 0)),
    )
    return pl.pallas_call(
        _cache_update_kernel,
        grid_spec=grid_spec,
        out_shape=jax.ShapeDtypeStruct((_B, _CANVAS, _D), cache.dtype),
    )(index, value, cache)


# speedup vs baseline: 48.3856x; 1.0105x over previous
import jax
import jax.numpy as jnp
from jax.experimental import pallas as pl
from jax.experimental.pallas import tpu as pltpu

_B = 32
_CHUNK = 128
_CANVAS = 8192
_D = 256
_RB = 8192
_ALIGN = 8
_WIN = _CHUNK + _ALIGN


def _cache_update_kernel(index_ref, value_ref, in_ref, out_ref):
    j = pl.program_id(1)
    start = j * _RB
    idx = index_ref[0]

    out_ref[...] = in_ref[...]

    overlap = jnp.logical_and(idx < start + _RB, idx + _CHUNK > start)

    @pl.when(overlap)
    def _merge():
        sh = idx - start
        p_raw = jnp.clip((sh // _ALIGN) * _ALIGN, 0, _RB - _WIN)
        p = pl.multiple_of(p_raw, _ALIGN)
        off = sh - p_raw
        padded = jnp.concatenate(
            [value_ref[...],
             jnp.zeros((1, _WIN - _CHUNK, _D), jnp.float32)], axis=1)
        rolled = pltpu.roll(padded, jnp.remainder(off, _WIN), axis=1)
        r = jax.lax.broadcasted_iota(jnp.int32, (1, _WIN, _D), 1)
        mask = jnp.logical_and(r >= off, r < off + _CHUNK)
        sub = in_ref[:, pl.ds(p, _WIN), :]
        out_ref[:, pl.ds(p, _WIN), :] = jnp.where(mask, rolled, sub)


def kernel(value, index, cache):
    grid_spec = pltpu.PrefetchScalarGridSpec(
        num_scalar_prefetch=1,
        grid=(_B, _CANVAS // _RB),
        in_specs=[
            pl.BlockSpec((1, _CHUNK, _D), lambda b, j, idx: (b, 0, 0)),
            pl.BlockSpec((1, _RB, _D), lambda b, j, idx: (b, j, 0)),
        ],
        out_specs=pl.BlockSpec((1, _RB, _D), lambda b, j, idx: (b, j, 0)),
    )
    return pl.pallas_call(
        _cache_update_kernel,
        grid_spec=grid_spec,
        out_shape=jax.ShapeDtypeStruct((_B, _CANVAS, _D), cache.dtype),
    )(index, value, cache)
